# Initial kernel scaffold; baseline (speedup 1.0000x reference)
import jax
import jax.numpy as jnp
from jax.experimental import pallas as pl


def kernel(g, features, W1, att_src, att_dst, W2):
    raise NotImplementedError("write your pallas kernel here")


# speedup vs baseline: 10.8068x; 10.8068x over previous
import jax
import jax.numpy as jnp
from jax import lax
from jax.experimental import pallas as pl
from jax.experimental.pallas import tpu as pltpu
from jax.experimental.pallas import tpu_sc as plsc

N = 10000
E = 320000
D = 128
NC = 2
NS = 16
L = 16
NW = NC * NS
EPT = 10368
EPAD = EPT * NW
CA = 96
CPTA = EPT // CA
CB = 128
CPTB = EPT // CB
NP = 10112
RPT = NP // NS
ND8 = 1280
DPT = ND8 // NS
KD = D // L


def _stage1_body(x_ref, w1_ref, ats_ref, atd_ref, h_ref, as_ref, ad_ref, m_ref):
    h = jnp.dot(x_ref[...], w1_ref[...], preferred_element_type=jnp.float32)
    h_ref[...] = h
    a_s = jnp.sum(h * ats_ref[...], axis=1, keepdims=True)
    a_d = jnp.sum(h * atd_ref[...], axis=1, keepdims=True)
    as_ref[...] = a_s
    ad_ref[...] = a_d
    m = jnp.max(a_s) + jnp.max(a_d)
    m = jnp.where(m >= 0.0, m, 0.2 * m)
    m_ref[...] = jnp.full((8, 128), m, jnp.float32)


def _stage1(x, w1, ats, atd):
    return pl.pallas_call(
        _stage1_body,
        out_shape=(
            jax.ShapeDtypeStruct((N, D), jnp.float32),
            jax.ShapeDtypeStruct((N, 1), jnp.float32),
            jax.ShapeDtypeStruct((N, 1), jnp.float32),
            jax.ShapeDtypeStruct((8, 128), jnp.float32),
        ),
    )(x, w1, ats, atd)


def _logit_body(gsrc, gdst, asrc, adst, m16, zr, w_out, outd,
                asrc_v, adst_v, m_v, src_c, dst_c, rowi_c, coli_c, w_t,
                den_st, acc_d):
    c = lax.axis_index("c")
    s = lax.axis_index("s")
    wid = c * NS + s

    pltpu.sync_copy(asrc, asrc_v)
    pltpu.sync_copy(adst, adst_v)
    pltpu.sync_copy(m16, m_v)

    dbase = s * DPT
    pltpu.sync_copy(zr.at[pl.ds(0, DPT)], acc_d.at[pl.ds(dbase, DPT)])
    pltpu.sync_copy(zr.at[pl.ds(0, CA)], den_st)

    m = m_v[pl.ds(0, L)]
    iota = lax.iota(jnp.int32, L)
    zv = jnp.zeros((L,), jnp.float32)
    ebase = wid * EPT

    plsc.subcore_barrier()

    def _chunk(j, carry):
        cbase = ebase + j * CA
        pltpu.sync_copy(gsrc.at[pl.ds(cbase, CA)], src_c)
        pltpu.sync_copy(gdst.at[pl.ds(cbase, CA)], dst_c)

        for k in range(CA // L):
            sv = src_c[pl.ds(k * L, L)]
            dv = dst_c[pl.ds(k * L, L)]
            a = plsc.load_gather(asrc_v, [sv])
            b = plsc.load_gather(adst_v, [dv])
            e = a + b
            e = jnp.where(e >= 0.0, e, 0.2 * e)
            w = jnp.exp(e - m)
            gid = cbase + k * L + iota
            w = jnp.where(gid < E, w, 0.0)
            w_t[pl.ds(j * CA + k * L, L)] = w
            rowi_c[pl.ds(k * L, L)] = lax.shift_right_logical(dv, 3)
            colv = lax.shift_left(jnp.bitwise_and(dv, 7), 4)
            coli_c[pl.ds(k * L, L)] = colv
            plsc.store_scatter(den_st, [iota + k * L, colv], w)

        pltpu.sync_copy(den_st, acc_d.at[rowi_c], add=True)

        for k in range(CA // L):
            cv = coli_c[pl.ds(k * L, L)]
            plsc.store_scatter(den_st, [iota + k * L, cv], zv)
        return carry

    lax.fori_loop(0, CPTA, _chunk, 0)

    pltpu.sync_copy(w_t, w_out.at[pl.ds(ebase, EPT)])
    plsc.subcore_barrier()
    pltpu.sync_copy(acc_d.at[pl.ds(dbase, DPT)], outd.at[c, pl.ds(dbase, DPT)])


def _kernel_a(gsrc, gdst, asrc, adst, m16, zr):
    mesh = plsc.VectorSubcoreMesh(core_axis_name="c", subcore_axis_name="s")
    fn = pl.kernel(
        _logit_body,
        out_type=(
            pltpu.HBM((EPAD,), jnp.float32),
            pltpu.HBM((NC, ND8, D), jnp.float32),
        ),
        mesh=mesh,
        compiler_params=pltpu.CompilerParams(needs_layout_passes=False),
        scratch_types=[
            pltpu.VMEM((N,), jnp.float32),
            pltpu.VMEM((N,), jnp.float32),
            pltpu.VMEM((L,), jnp.float32),
            pltpu.VMEM((CA,), jnp.int32),
            pltpu.VMEM((CA,), jnp.int32),
            pltpu.VMEM((CA,), jnp.int32),
            pltpu.VMEM((CA,), jnp.int32),
            pltpu.VMEM((EPT,), jnp.float32),
            pltpu.VMEM((CA, D), jnp.float32),
            pltpu.VMEM_SHARED((ND8, D), jnp.float32),
        ],
    )
    return fn(gsrc, gdst, asrc, adst, m16, zr)


def _agg_body(gsrc, gdst, h, w_all, zr, outp,
              src_c, dst_c, w_c, rows_v, acc_p, sem):
    c = lax.axis_index("c")
    s = lax.axis_index("s")
    wid = c * NS + s

    rbase = s * RPT
    pltpu.sync_copy(zr.at[pl.ds(0, RPT)], acc_p.at[pl.ds(rbase, RPT)])

    zi = jnp.zeros((L,), jnp.int32)
    ebase = wid * EPT

    plsc.subcore_barrier()

    def _chunk(j, carry):
        cbase = ebase + j * CB
        pltpu.sync_copy(gsrc.at[pl.ds(cbase, CB)], src_c)
        cp = pltpu.async_copy(h.at[src_c], rows_v, sem)
        pltpu.sync_copy(gdst.at[pl.ds(cbase, CB)], dst_c)
        pltpu.sync_copy(w_all.at[pl.ds(cbase, CB)], w_c)
        cp.wait()

        def _weight(e, inner):
            ws = plsc.load_gather(w_c, [zi + e])
            for k in range(KD):
                rows_v[e, pl.ds(k * L, L)] = rows_v[e, pl.ds(k * L, L)] * ws
            return inner

        lax.fori_loop(0, CB, _weight, 0)

        pltpu.sync_copy(rows_v, acc_p.at[dst_c], add=True)
        return carry

    lax.fori_loop(0, CPTB, _chunk, 0)

    plsc.subcore_barrier()
    pltpu.sync_copy(acc_p.at[pl.ds(rbase, RPT)], outp.at[c, pl.ds(rbase, RPT)])


def _kernel_b(gsrc, gdst, h, w_all, zr):
    mesh = plsc.VectorSubcoreMesh(core_axis_name="c", subcore_axis_name="s")
    fn = pl.kernel(
        _agg_body,
        out_type=pltpu.HBM((NC, NP, D), jnp.float32),
        mesh=mesh,
        compiler_params=pltpu.CompilerParams(needs_layout_passes=False),
        scratch_types=[
            pltpu.VMEM((CB,), jnp.int32),
            pltpu.VMEM((CB,), jnp.int32),
            pltpu.VMEM((CB,), jnp.float32),
            pltpu.VMEM((CB, D), jnp.float32),
            pltpu.VMEM_SHARED((NP, D), jnp.float32),
            pltpu.SemaphoreType.DMA,
        ],
    )
    return fn(gsrc, gdst, h, w_all, zr)


def _stage4_body(p_ref, den_ref, w2_ref, o_ref):
    x = (p_ref[0, :N] + p_ref[1, :N]) / (den_ref[:N] + 1e-16)
    x = jnp.where(x > 0.0, x, jnp.exp(x) - 1.0)
    o_ref[...] = jnp.dot(x, w2_ref[...], preferred_element_type=jnp.float32)


def _stage4(outp, den, w2):
    return pl.pallas_call(
        _stage4_body,
        out_shape=jax.ShapeDtypeStruct((N, D), jnp.float32),
    )(outp, den, w2)


@jax.jit
def kernel(g, features, W1, att_src, att_dst, W2):
    h, a_s, a_d, m8 = _stage1(
        features, W1, att_src.reshape(1, D), att_dst.reshape(1, D)
    )
    asrc = a_s.reshape(N)
    adst = a_d.reshape(N)
    m16 = m8[0, :L]

    gp = jnp.concatenate(
        [g.astype(jnp.int32), jnp.zeros((2, EPAD - E), jnp.int32)], axis=1
    )
    zr = jnp.zeros((RPT, D), jnp.float32)

    w_all, outd = _kernel_a(gp[0], gp[1], asrc, adst, m16, zr)
    outp = _kernel_b(gp[0], gp[1], h, w_all, zr)

    den = (outd[0, :, ::16] + outd[1, :, ::16]).reshape(ND8 * 8, 1)

    return _stage4(outp, den, W2)


# speedup vs baseline: 13.5063x; 1.2498x over previous
import jax
import jax.numpy as jnp
from jax import lax
from jax.experimental import pallas as pl
from jax.experimental.pallas import tpu as pltpu
from jax.experimental.pallas import tpu_sc as plsc

N = 10000
E = 320000
D = 128
NC = 2
NS = 16
L = 16
NW = NC * NS
EPT = 10368
EPAD = EPT * NW
CA = 96
CPTA = EPT // CA
CB = 96
CPTB = EPT // CB
UNROLL = 4
NP = 10112
RPT = NP // NS
ND8 = 1280
DPT = ND8 // NS
KD = D // L


def _stage1_body(x_ref, w1_ref, ats_ref, atd_ref, h_ref, as_ref, ad_ref, m_ref):
    h = jnp.dot(x_ref[...], w1_ref[...], preferred_element_type=jnp.float32)
    h_ref[...] = h
    a_s = jnp.sum(h * ats_ref[...], axis=1, keepdims=True)
    a_d = jnp.sum(h * atd_ref[...], axis=1, keepdims=True)
    as_ref[...] = a_s
    ad_ref[...] = a_d
    m = jnp.max(a_s) + jnp.max(a_d)
    m = jnp.where(m >= 0.0, m, 0.2 * m)
    m_ref[...] = jnp.full((8, 128), m, jnp.float32)


def _stage1(x, w1, ats, atd):
    return pl.pallas_call(
        _stage1_body,
        out_shape=(
            jax.ShapeDtypeStruct((N, D), jnp.float32),
            jax.ShapeDtypeStruct((N, 1), jnp.float32),
            jax.ShapeDtypeStruct((N, 1), jnp.float32),
            jax.ShapeDtypeStruct((8, 128), jnp.float32),
        ),
    )(x, w1, ats, atd)


def _logit_body(gsrc, gdst, asrc, adst, m16, zr, w_out, outd,
                asrc_v, adst_v, m_v, src_c, dst_c, rowi_c, coli_c, w_t,
                den_st, acc_d):
    c = lax.axis_index("c")
    s = lax.axis_index("s")
    wid = c * NS + s

    pltpu.sync_copy(asrc, asrc_v)
    pltpu.sync_copy(adst, adst_v)
    pltpu.sync_copy(m16, m_v)

    dbase = s * DPT
    pltpu.sync_copy(zr.at[pl.ds(0, DPT)], acc_d.at[pl.ds(dbase, DPT)])
    pltpu.sync_copy(zr.at[pl.ds(0, CA)], den_st)

    m = m_v[pl.ds(0, L)]
    iota = lax.iota(jnp.int32, L)
    zv = jnp.zeros((L,), jnp.float32)
    ebase = wid * EPT

    plsc.subcore_barrier()

    def _chunk(j, carry):
        cbase = ebase + j * CA
        pltpu.sync_copy(gsrc.at[pl.ds(cbase, CA)], src_c)
        pltpu.sync_copy(gdst.at[pl.ds(cbase, CA)], dst_c)

        for k in range(CA // L):
            sv = src_c[pl.ds(k * L, L)]
            dv = dst_c[pl.ds(k * L, L)]
            a = plsc.load_gather(asrc_v, [sv])
            b = plsc.load_gather(adst_v, [dv])
            e = a + b
            e = jnp.where(e >= 0.0, e, 0.2 * e)
            w = jnp.exp(e - m)
            gid = cbase + k * L + iota
            w = jnp.where(gid < E, w, 0.0)
            w_t[pl.ds(j * CA + k * L, L)] = w
            rowi_c[pl.ds(k * L, L)] = lax.shift_right_logical(dv, 3)
            colv = lax.shift_left(jnp.bitwise_and(dv, 7), 4)
            coli_c[pl.ds(k * L, L)] = colv
            plsc.store_scatter(den_st, [iota + k * L, colv], w)

        pltpu.sync_copy(den_st, acc_d.at[rowi_c], add=True)

        for k in range(CA // L):
            cv = coli_c[pl.ds(k * L, L)]
            plsc.store_scatter(den_st, [iota + k * L, cv], zv)
        return carry

    lax.fori_loop(0, CPTA, _chunk, 0)

    pltpu.sync_copy(w_t, w_out.at[pl.ds(ebase, EPT)])
    plsc.subcore_barrier()
    pltpu.sync_copy(acc_d.at[pl.ds(dbase, DPT)], outd.at[c, pl.ds(dbase, DPT)])


def _kernel_a(gsrc, gdst, asrc, adst, m16, zr):
    mesh = plsc.VectorSubcoreMesh(core_axis_name="c", subcore_axis_name="s")
    fn = pl.kernel(
        _logit_body,
        out_type=(
            pltpu.HBM((EPAD,), jnp.float32),
            pltpu.HBM((NC, ND8, D), jnp.float32),
        ),
        mesh=mesh,
        compiler_params=pltpu.CompilerParams(needs_layout_passes=False),
        scratch_types=[
            pltpu.VMEM((N,), jnp.float32),
            pltpu.VMEM((N,), jnp.float32),
            pltpu.VMEM((L,), jnp.float32),
            pltpu.VMEM((CA,), jnp.int32),
            pltpu.VMEM((CA,), jnp.int32),
            pltpu.VMEM((CA,), jnp.int32),
            pltpu.VMEM((CA,), jnp.int32),
            pltpu.VMEM((EPT,), jnp.float32),
            pltpu.VMEM((CA, D), jnp.float32),
            pltpu.VMEM_SHARED((ND8, D), jnp.float32),
        ],
    )
    return fn(gsrc, gdst, asrc, adst, m16, zr)


def _agg_body(gsrc, gdst, h, w_all, zr, outp,
              src0, src1, dst0, dst1, w0, w1, rows0, rows1, acc_p,
              gs0, gs1):
    c = lax.axis_index("c")
    s = lax.axis_index("s")
    wid = c * NS + s

    rbase = s * RPT
    pltpu.sync_copy(zr.at[pl.ds(0, RPT)], acc_p.at[pl.ds(rbase, RPT)])

    zi = jnp.zeros((L,), jnp.int32)
    ebase = wid * EPT

    plsc.subcore_barrier()

    def _process(j, src_c, dst_c, w_c, rows_v, gsem):
        cbase = ebase + j * CB
        pltpu.sync_copy(gdst.at[pl.ds(cbase, CB)], dst_c)
        pltpu.sync_copy(w_all.at[pl.ds(cbase, CB)], w_c)
        pltpu.make_async_copy(h.at[src_c], rows_v, gsem).wait()

        def _weight(e, inner):
            for u in range(UNROLL):
                eu = e * UNROLL + u
                ws = plsc.load_gather(w_c, [zi + eu])
                for k in range(KD):
                    rows_v[eu, pl.ds(k * L, L)] = (
                        rows_v[eu, pl.ds(k * L, L)] * ws
                    )
            return inner

        lax.fori_loop(0, CB // UNROLL, _weight, 0)
        pltpu.sync_copy(rows_v, acc_p.at[dst_c], add=True)

    def _start(j, src_c, rows_v, gsem):
        pltpu.sync_copy(gsrc.at[pl.ds(ebase + j * CB, CB)], src_c)
        pltpu.async_copy(h.at[src_c], rows_v, gsem)

    _start(0, src0, rows0, gs0)
    _start(1, src1, rows1, gs1)

    def _pair(i, carry):
        _process(2 * i, src0, dst0, w0, rows0, gs0)

        @pl.when(i < CPTB // 2 - 1)
        def _():
            _start(2 * i + 2, src0, rows0, gs0)

        _process(2 * i + 1, src1, dst1, w1, rows1, gs1)

        @pl.when(i < CPTB // 2 - 1)
        def _():
            _start(2 * i + 3, src1, rows1, gs1)

        return carry

    lax.fori_loop(0, CPTB // 2, _pair, 0)

    plsc.subcore_barrier()
    pltpu.sync_copy(acc_p.at[pl.ds(rbase, RPT)], outp.at[c, pl.ds(rbase, RPT)])


def _kernel_b(gsrc, gdst, h, w_all, zr):
    mesh = plsc.VectorSubcoreMesh(core_axis_name="c", subcore_axis_name="s")
    fn = pl.kernel(
        _agg_body,
        out_type=pltpu.HBM((NC, NP, D), jnp.float32),
        mesh=mesh,
        compiler_params=pltpu.CompilerParams(needs_layout_passes=False),
        scratch_types=[
            pltpu.VMEM((CB,), jnp.int32),
            pltpu.VMEM((CB,), jnp.int32),
            pltpu.VMEM((CB,), jnp.int32),
            pltpu.VMEM((CB,), jnp.int32),
            pltpu.VMEM((CB,), jnp.float32),
            pltpu.VMEM((CB,), jnp.float32),
            pltpu.VMEM((CB, D), jnp.float32),
            pltpu.VMEM((CB, D), jnp.float32),
            pltpu.VMEM_SHARED((NP, D), jnp.float32),
            pltpu.SemaphoreType.DMA,
            pltpu.SemaphoreType.DMA,
        ],
    )
    return fn(gsrc, gdst, h, w_all, zr)


def _stage4_body(p_ref, den_ref, w2_ref, o_ref):
    x = (p_ref[0, :N] + p_ref[1, :N]) / (den_ref[:N] + 1e-16)
    x = jnp.where(x > 0.0, x, jnp.exp(x) - 1.0)
    o_ref[...] = jnp.dot(x, w2_ref[...], preferred_element_type=jnp.float32)


def _stage4(outp, den, w2):
    return pl.pallas_call(
        _stage4_body,
        out_shape=jax.ShapeDtypeStruct((N, D), jnp.float32),
    )(outp, den, w2)


@jax.jit
def kernel(g, features, W1, att_src, att_dst, W2):
    h, a_s, a_d, m8 = _stage1(
        features, W1, att_src.reshape(1, D), att_dst.reshape(1, D)
    )
    asrc = a_s.reshape(N)
    adst = a_d.reshape(N)
    m16 = m8[0, :L]

    gp = jnp.concatenate(
        [g.astype(jnp.int32), jnp.zeros((2, EPAD - E), jnp.int32)], axis=1
    )
    zr = jnp.zeros((RPT, D), jnp.float32)

    w_all, outd = _kernel_a(gp[0], gp[1], asrc, adst, m16, zr)
    outp = _kernel_b(gp[0], gp[1], h, w_all, zr)

    den = (outd[0, :, ::16] + outd[1, :, ::16]).reshape(ND8 * 8, 1)

    return _stage4(outp, den, W2)
